# baseline (speedup 1.0000x reference)
import functools

import jax
import jax.numpy as jnp
from jax import lax
from jax.experimental import pallas as pl
from jax.experimental.pallas import tpu as pltpu
from jax.experimental.pallas import tpu_sc as plsc

B = 16384
D = 128
DIM_OUT = 1024
NC, NS = 2, 16
NW = NC * NS
CHUNK = 128

SLICES = 4
BS = B // SLICES
BPW = BS // NW
NCHUNK = BPW // CHUNK

_sc_mesh = plsc.VectorSubcoreMesh(core_axis_name="c", subcore_axis_name="s")


@functools.partial(
    pl.kernel,
    out_type=(
        jax.ShapeDtypeStruct((BS // CHUNK, CHUNK, D), jnp.float32),
        jax.ShapeDtypeStruct((BS // CHUNK, CHUNK, D), jnp.float32),
    ),
    mesh=_sc_mesh,
    scratch_types=[
        pltpu.VMEM((NCHUNK, CHUNK), jnp.int32),
        pltpu.VMEM((NCHUNK, CHUNK), jnp.int32),
        pltpu.VMEM((NCHUNK, CHUNK, D), jnp.float32),
        pltpu.VMEM((NCHUNK, CHUNK, D), jnp.float32),
        pltpu.SemaphoreType.DMA,
    ],
)
def _sc_gather(c0_hbm, c1_hbm, t0_hbm, t1_hbm, e0_hbm, e1_hbm,
               idx0_v, idx1_v, rows0_v, rows1_v, sem):
    wid = lax.axis_index("s") * NC + lax.axis_index("c")
    cbase = wid * NCHUNK
    pltpu.sync_copy(c0_hbm.at[pl.ds(cbase, NCHUNK)], idx0_v)
    pltpu.sync_copy(c1_hbm.at[pl.ds(cbase, NCHUNK)], idx1_v)
    for j in range(NCHUNK):
        pltpu.async_copy(t0_hbm.at[idx0_v.at[j]], rows0_v.at[j], sem)
        pltpu.async_copy(t1_hbm.at[idx1_v.at[j]], rows1_v.at[j], sem)
    for j in range(NCHUNK):
        pltpu.make_async_copy(t0_hbm.at[idx0_v.at[j]], rows0_v.at[j], sem).wait()
        pltpu.make_async_copy(t1_hbm.at[idx1_v.at[j]], rows1_v.at[j], sem).wait()
    pltpu.sync_copy(rows0_v, e0_hbm.at[pl.ds(cbase, NCHUNK)])
    pltpu.sync_copy(rows1_v, e1_hbm.at[pl.ds(cbase, NCHUNK)])


BLK = 2048
SPB = BS // BLK


def _mlp_body(e0_ref, e1_ref, w0_ref, w1_ref, b_ref, o_ref):
    e0 = e0_ref[...].astype(jnp.bfloat16)
    e1 = e1_ref[...].astype(jnp.bfloat16)
    h = jnp.dot(e0, w0_ref[...], preferred_element_type=jnp.float32)
    h = h + jnp.dot(e1, w1_ref[...], preferred_element_type=jnp.float32)
    h = h + b_ref[...]
    o_ref[...] = h * jax.nn.sigmoid(h)


def _mlp_body_aliased(_acc_ref, e0_ref, e1_ref, w0_ref, w1_ref, b_ref, o_ref):
    _mlp_body(e0_ref, e1_ref, w0_ref, w1_ref, b_ref, o_ref)


_in_specs_common = [
    pl.BlockSpec((BLK, D), lambda i: (i, 0)),
    pl.BlockSpec((BLK, D), lambda i: (i, 0)),
    pl.BlockSpec((D, DIM_OUT), lambda i: (0, 0)),
    pl.BlockSpec((D, DIM_OUT), lambda i: (0, 0)),
    pl.BlockSpec((1, DIM_OUT), lambda i: (0, 0)),
]

_mlp_first = pl.pallas_call(
    _mlp_body,
    grid=(SPB,),
    in_specs=_in_specs_common,
    out_specs=pl.BlockSpec((BLK, DIM_OUT), lambda i: (i, 0)),
    out_shape=jax.ShapeDtypeStruct((B, DIM_OUT), jnp.float32),
)

_mlp_slice = []
for _s in range(1, SLICES):
    _mlp_slice.append(pl.pallas_call(
        _mlp_body_aliased,
        grid=(SPB,),
        in_specs=[pl.BlockSpec(memory_space=pl.ANY)] + _in_specs_common,
        out_specs=pl.BlockSpec((BLK, DIM_OUT),
                               lambda i, _s=_s: (_s * SPB + i, 0)),
        out_shape=jax.ShapeDtypeStruct((B, DIM_OUT), jnp.float32),
        input_output_aliases={0: 0},
    ))


@jax.jit
def kernel(cond, cond_embed0, cond_embed1, W, b):
    c0 = cond[:, 0].reshape(B // CHUNK, CHUNK)
    c1 = cond[:, 1].reshape(B // CHUNK, CHUNK)
    Wb = W.astype(jnp.bfloat16)
    W0, W1 = Wb[:D], Wb[D:]
    b2 = b.reshape(1, DIM_OUT)
    rps = BS // CHUNK
    es = []
    for s in range(SLICES):
        e0, e1 = _sc_gather(
            lax.slice_in_dim(c0, s * rps, (s + 1) * rps),
            lax.slice_in_dim(c1, s * rps, (s + 1) * rps),
            cond_embed0, cond_embed1)
        es.append((e0.reshape(BS, D), e1.reshape(BS, D)))
    out = _mlp_first(es[0][0], es[0][1], W0, W1, b2)
    for s in range(1, SLICES):
        out = _mlp_slice[s - 1](out, es[s][0], es[s][1], W0, W1, b2)
    return out
